# baseline (speedup 1.0000x reference)
import functools

import jax
import jax.numpy as jnp
from jax.experimental import pallas as pl
from jax.experimental.pallas import tpu as pltpu

S = 2048
M = 768
DFF = 3072
TOKBLK = 256
NTOK = S // TOKBLK
NW1 = 4
DFFBLK = DFF // NW1
LAST = (NW1 - 1) * DFFBLK


def _ffn_kernel(
    x_ref, w1_ref, b1_ref, w2_ref, b2_ref, out_ref,
    xb_ref, h_ref, yacc_ref, w1v_ref, w2v_ref, w1b_ref, w2b_ref, sem1, sem2,
):
    step = pl.program_id(0)

    @pl.when(step == 0)
    def _start():
        for j in range(NW1):
            pltpu.make_async_copy(
                w1_ref.at[:, j * DFFBLK:(j + 1) * DFFBLK],
                w1v_ref.at[:, j * DFFBLK:(j + 1) * DFFBLK],
                sem1.at[j],
            ).start()
            pltpu.make_async_copy(
                w2_ref.at[j * DFFBLK:(j + 1) * DFFBLK, :],
                w2v_ref.at[j * DFFBLK:(j + 1) * DFFBLK, :],
                sem2.at[j],
            ).start()
        xb_ref[...] = x_ref[...].astype(jnp.bfloat16)

    for k in range(NW1):
        @pl.when(step == k)
        def _h_phase(k=k):
            lo, hi = k * DFFBLK, (k + 1) * DFFBLK
            pltpu.make_async_copy(
                w1_ref.at[:, lo:hi], w1v_ref.at[:, lo:hi], sem1.at[k]
            ).wait()
            w1b_ref[...] = w1v_ref[:, lo:hi].astype(jnp.bfloat16)
            h = jnp.dot(
                xb_ref[...], w1b_ref[...], preferred_element_type=jnp.float32
            )
            h = jnp.maximum(h + b1_ref[:, lo:hi], 0.0)
            h_ref[:, lo:hi] = h.astype(jnp.bfloat16)

    for k in range(NW1 - 1):
        @pl.when(step == k + 1)
        def _y_accum(k=k):
            lo, hi = k * DFFBLK, (k + 1) * DFFBLK
            pltpu.make_async_copy(
                w2_ref.at[lo:hi, :], w2v_ref.at[lo:hi, :], sem2.at[k]
            ).wait()
            w2b_ref[lo:hi, :] = w2v_ref[lo:hi, :].astype(jnp.bfloat16)
            y = jnp.dot(
                h_ref[:, lo:hi],
                w2b_ref[lo:hi, :],
                preferred_element_type=jnp.float32,
            )
            if k == 0:
                yacc_ref[...] = y + b2_ref[...]
            else:
                yacc_ref[...] = yacc_ref[...] + y

    @pl.when(step == NW1)
    def _w2_last():
        pltpu.make_async_copy(
            w2_ref.at[LAST:DFF, :], w2v_ref.at[LAST:DFF, :], sem2.at[NW1 - 1]
        ).wait()
        w2b_ref[LAST:DFF, :] = w2v_ref[LAST:DFF, :].astype(jnp.bfloat16)

    for t in range(NTOK):
        @pl.when(step == NW1 + t)
        def _y_tail(t=t):
            t0 = t * TOKBLK
            y = jnp.dot(
                h_ref[t0:t0 + TOKBLK, LAST:DFF],
                w2b_ref[LAST:DFF, :],
                preferred_element_type=jnp.float32,
            )
            out_ref[...] = yacc_ref[t0:t0 + TOKBLK, :] + y


@functools.partial(jax.jit, static_argnames=())
def kernel(inputs, Wg, bg, W1, b1, W2, b2):
    x = inputs.reshape(-1, M)

    out = pl.pallas_call(
        _ffn_kernel,
        grid=(NW1 + NTOK,),
        out_shape=jax.ShapeDtypeStruct((S, M), jnp.float32),
        in_specs=[
            pl.BlockSpec((S, M), lambda i: (0, 0)),
            pl.BlockSpec(memory_space=pl.ANY),
            pl.BlockSpec((1, DFF), lambda i: (0, 0)),
            pl.BlockSpec(memory_space=pl.ANY),
            pl.BlockSpec((1, M), lambda i: (0, 0)),
        ],
        out_specs=pl.BlockSpec(
            (TOKBLK, M), lambda i: (jnp.maximum(i - NW1, 0), 0)
        ),
        scratch_shapes=[
            pltpu.VMEM((S, M), jnp.bfloat16),
            pltpu.VMEM((S, DFF), jnp.bfloat16),
            pltpu.VMEM((S, M), jnp.float32),
            pltpu.VMEM((M, DFF), jnp.float32),
            pltpu.VMEM((DFF, M), jnp.float32),
            pltpu.VMEM((M, DFFBLK), jnp.bfloat16),
            pltpu.VMEM((DFF, M), jnp.bfloat16),
            pltpu.SemaphoreType.DMA((NW1,)),
            pltpu.SemaphoreType.DMA((NW1,)),
        ],
        compiler_params=pltpu.CompilerParams(
            vmem_limit_bytes=120 * 1024 * 1024,
        ),
    )(x, W1, b1.reshape(1, DFF), W2, b2.reshape(1, M))

    return out.reshape(inputs.shape)
